# Initial kernel scaffold; baseline (speedup 1.0000x reference)
import jax
import jax.numpy as jnp
from jax.experimental import pallas as pl


def kernel(object_features, edge_index, W_emb, b_emb, Wr0, br0, Wo0, Wr1, br1, Wo1, W_head, b_head):
    raise NotImplementedError("write your pallas kernel here")


# speedup vs baseline: 4.9588x; 4.9588x over previous
import functools

import jax
import jax.numpy as jnp
from jax import lax
from jax.experimental import pallas as pl
from jax.experimental.pallas import tpu as pltpu
from jax.experimental.pallas import tpu_sc as plsc

N = 10000
E = 320000
H = 128
C = 51
CP = 64

NC = 2
NS = 16
NW = NC * NS
L = 16

N_PAD = 10240
ROWS_PER_TILE = N_PAD // NS
EPW = E // NW
K = 128
NCH = EPW // K
REM = EPW - NCH * K


def _vsc_mesh():
    return plsc.VectorSubcoreMesh(core_axis_name="c", subcore_axis_name="s")


@functools.partial(
    pl.kernel,
    out_type=jax.ShapeDtypeStruct((NC, N_PAD, H), jnp.float32),
    mesh=_vsc_mesh(),
    scratch_types=[
        pltpu.VMEM((K,), jnp.int32),
        pltpu.VMEM((K,), jnp.int32),
        pltpu.VMEM((K, H), jnp.float32),
        pltpu.VMEM((REM,), jnp.int32),
        pltpu.VMEM((REM,), jnp.int32),
        pltpu.VMEM((REM, H), jnp.float32),
        pltpu.VMEM_SHARED((N_PAD, H), jnp.float32),
        pltpu.SemaphoreType.DMA,
    ],
)
def _sc_segment_sum(x_hbm, src_hbm, dst_hbm, zeros_hbm, out_hbm,
                    sidx, didx, rows, sidx_r, didx_r, rows_r, agg_sh, sem):
    c = lax.axis_index("c")
    s = lax.axis_index("s")
    wid = s * NC + c
    row0 = s * ROWS_PER_TILE
    pltpu.sync_copy(zeros_hbm, agg_sh.at[pl.ds(row0, ROWS_PER_TILE)])
    plsc.subcore_barrier()

    ebase = wid * EPW

    def chunk(i, carry):
        off = ebase + i * K
        pltpu.sync_copy(src_hbm.at[pl.ds(off, K)], sidx)
        pltpu.sync_copy(dst_hbm.at[pl.ds(off, K)], didx)
        pltpu.async_copy(x_hbm.at[sidx], rows, sem).wait()
        pltpu.sync_copy(rows, agg_sh.at[didx], add=True)
        return carry

    lax.fori_loop(0, NCH, chunk, 0)

    off = ebase + NCH * K
    pltpu.sync_copy(src_hbm.at[pl.ds(off, REM)], sidx_r)
    pltpu.sync_copy(dst_hbm.at[pl.ds(off, REM)], didx_r)
    pltpu.async_copy(x_hbm.at[sidx_r], rows_r, sem).wait()
    pltpu.sync_copy(rows_r, agg_sh.at[didx_r], add=True)

    plsc.subcore_barrier()
    pltpu.sync_copy(agg_sh.at[pl.ds(row0, ROWS_PER_TILE)],
                    out_hbm.at[c, pl.ds(row0, ROWS_PER_TILE)])


@functools.partial(
    pl.kernel,
    out_type=jax.ShapeDtypeStruct((E, CP), jnp.float32),
    mesh=_vsc_mesh(),
    compiler_params=pltpu.CompilerParams(use_tc_tiling_on_sc=False),
    scratch_types=[
        pltpu.VMEM((K,), jnp.int32),
        pltpu.VMEM((K,), jnp.int32),
        pltpu.VMEM((K, CP), jnp.float32),
        pltpu.VMEM((K, CP), jnp.float32),
        pltpu.VMEM((REM,), jnp.int32),
        pltpu.VMEM((REM,), jnp.int32),
        pltpu.VMEM((REM, CP), jnp.float32),
        pltpu.VMEM((REM, CP), jnp.float32),
        pltpu.SemaphoreType.DMA,
    ],
)
def _sc_head(psrc_hbm, pdst_hbm, src_hbm, dst_hbm, out_hbm,
             sidx, didx, ra, rb, sidx_r, didx_r, ra_r, rb_r, sem):
    c = lax.axis_index("c")
    s = lax.axis_index("s")
    wid = s * NC + c
    ebase = wid * EPW
    ncol = CP // L

    def add_rows(a, b, nrows):
        def row_body(r, carry):
            for j in range(ncol):
                sl = pl.ds(j * L, L)
                a[r, sl] = a[r, sl] + b[r, sl]
            return carry
        lax.fori_loop(0, nrows, row_body, 0)

    def chunk(i, carry):
        off = ebase + i * K
        pltpu.sync_copy(src_hbm.at[pl.ds(off, K)], sidx)
        pltpu.sync_copy(dst_hbm.at[pl.ds(off, K)], didx)
        pltpu.async_copy(psrc_hbm.at[sidx], ra, sem).wait()
        pltpu.async_copy(pdst_hbm.at[didx], rb, sem).wait()
        add_rows(ra, rb, K)
        pltpu.sync_copy(ra, out_hbm.at[pl.ds(off, K)])
        return carry

    lax.fori_loop(0, NCH, chunk, 0)

    off = ebase + NCH * K
    pltpu.sync_copy(src_hbm.at[pl.ds(off, REM)], sidx_r)
    pltpu.sync_copy(dst_hbm.at[pl.ds(off, REM)], didx_r)
    pltpu.async_copy(psrc_hbm.at[sidx_r], ra_r, sem).wait()
    pltpu.async_copy(pdst_hbm.at[didx_r], rb_r, sem).wait()
    add_rows(ra_r, rb_r, REM)
    pltpu.sync_copy(ra_r, out_hbm.at[pl.ds(off, REM)])


def _tc_embed(obj, w, b):
    def body(x_ref, w_ref, b_ref, o_ref):
        o_ref[...] = jnp.maximum(
            jnp.dot(x_ref[...], w_ref[...], preferred_element_type=jnp.float32)
            + b_ref[...], 0.0)
    return pl.pallas_call(
        body, out_shape=jax.ShapeDtypeStruct((N, H), jnp.float32),
    )(obj, w, b)


def _tc_layer(agg, x, wr, wo, br):
    def body(agg_ref, x_ref, wr_ref, wo_ref, br_ref, o_ref):
        a = agg_ref[0, :N, :] + agg_ref[1, :N, :]
        o_ref[...] = jnp.maximum(
            jnp.dot(a, wr_ref[...], preferred_element_type=jnp.float32)
            + jnp.dot(x_ref[...], wo_ref[...], preferred_element_type=jnp.float32)
            + br_ref[...], 0.0)
    return pl.pallas_call(
        body, out_shape=jax.ShapeDtypeStruct((N, H), jnp.float32),
    )(agg, x, wr, wo, br)


def _tc_layer_head(agg, x, wr, wo, br, ws, wd, bh):
    def body(agg_ref, x_ref, wr_ref, wo_ref, br_ref, ws_ref, wd_ref, bh_ref,
             ps_ref, pd_ref):
        a = agg_ref[0, :N, :] + agg_ref[1, :N, :]
        x2 = jnp.maximum(
            jnp.dot(a, wr_ref[...], preferred_element_type=jnp.float32)
            + jnp.dot(x_ref[...], wo_ref[...], preferred_element_type=jnp.float32)
            + br_ref[...], 0.0)
        ps_ref[...] = jnp.dot(x2, ws_ref[...],
                              preferred_element_type=jnp.float32) + bh_ref[...]
        pd_ref[...] = jnp.dot(x2, wd_ref[...],
                              preferred_element_type=jnp.float32)
    return pl.pallas_call(
        body,
        out_shape=(jax.ShapeDtypeStruct((N, CP), jnp.float32),
                   jax.ShapeDtypeStruct((N, CP), jnp.float32)),
    )(agg, x, wr, wo, br, ws, wd, bh)


def kernel(object_features, edge_index, W_emb, b_emb, Wr0, br0, Wo0,
           Wr1, br1, Wo1, W_head, b_head):
    src = edge_index[0]
    dst = edge_index[1]
    zeros = jnp.zeros((ROWS_PER_TILE, H), jnp.float32)

    x = _tc_embed(object_features, W_emb, b_emb.reshape(1, H))

    agg0 = _sc_segment_sum(x, src, dst, zeros)
    x1 = _tc_layer(agg0, x, Wr0, Wo0, br0.reshape(1, H))

    ws = jnp.pad(W_head[:H], ((0, 0), (0, CP - C)))
    wd = jnp.pad(W_head[H:], ((0, 0), (0, CP - C)))
    bh = jnp.pad(b_head, (0, CP - C)).reshape(1, CP)

    agg1 = _sc_segment_sum(x1, src, dst, zeros)
    psrc, pdst = _tc_layer_head(agg1, x1, Wr1, Wo1, br1.reshape(1, H),
                                ws, wd, bh)

    out = _sc_head(psrc, pdst, src, dst)
    return out[:, :C]
